# Initial kernel scaffold; baseline (speedup 1.0000x reference)
import jax
import jax.numpy as jnp
from jax.experimental import pallas as pl


def kernel(x, edge_attr, edge_index, batch, static_features, emb_node_0, emb_node_1, emb_node_2, emb_node_3, emb_edge_0, emb_edge_1, emb_edge_2, W_in, b_in, Wself_0, Wnbr_0, Wedge_0, b_0, Wself_1, Wnbr_1, Wedge_1, b_1, Wself_2, Wnbr_2, Wedge_2, b_2, We1, be1, We2, be2, Wp1, bp1, Wp2, bp2):
    raise NotImplementedError("write your pallas kernel here")


# speedup vs baseline: 4.7775x; 4.7775x over previous
import functools
import jax
import jax.numpy as jnp
from jax import lax
from jax.experimental import pallas as pl
from jax.experimental.pallas import tpu as pltpu

HID = 128
EMB = 32
G = 32
STATIC = 16
EDGE_DIM = 8
DEPTH = 3

ROWB = 1000
ECHUNK = 512


def _emb_body(xh_ref, t_ref, o_ref):
    xh = xh_ref[...]
    tab = t_ref[...]
    iota = lax.broadcasted_iota(jnp.int32, (ROWB, 128), 1)
    ne = jnp.dot((xh[:, 0:1] == iota).astype(jnp.float32), tab,
                 preferred_element_type=jnp.float32, precision=lax.Precision.HIGHEST)
    for j in range(1, 4):
        oh = (xh[:, j:j + 1] == iota).astype(jnp.float32)
        ne = ne + jnp.dot(oh, tab, preferred_element_type=jnp.float32,
                          precision=lax.Precision.HIGHEST)
    o_ref[...] = ne


def _emb(xh, t_all, n):
    return pl.pallas_call(
        _emb_body,
        grid=(n // ROWB,),
        in_specs=[
            pl.BlockSpec((ROWB, 4), lambda i: (i, 0)),
            pl.BlockSpec((128, EMB), lambda i: (0, 0)),
        ],
        out_specs=pl.BlockSpec((ROWB, EMB), lambda i: (i, 0)),
        out_shape=jax.ShapeDtypeStruct((n, EMB), jnp.float32),
    )(xh, t_all)


def _mm_body(x_ref, w_ref, o_ref):
    o_ref[...] = jnp.dot(x_ref[...], w_ref[...], preferred_element_type=jnp.float32)


def _mm(x, w, n):
    return pl.pallas_call(
        _mm_body,
        grid=(n // ROWB,),
        in_specs=[pl.BlockSpec((ROWB, HID), lambda i: (i, 0)),
                  pl.BlockSpec((HID, HID), lambda i: (0, 0))],
        out_specs=pl.BlockSpec((ROWB, HID), lambda i: (i, 0)),
        out_shape=jax.ShapeDtypeStruct((n, HID), jnp.float32),
    )(x, w)


def _layer_body(x_ref, h0_ref, h1_ref, ws_ref, wn_ref, b_ref, o_ref, on_ref):
    x = x_ref[...]
    h = h0_ref[...] + h1_ref[...]
    z = (jnp.dot(x, ws_ref[...], preferred_element_type=jnp.float32)
         + h + b_ref[...])
    xo = x + jnp.maximum(z, 0.0)
    o_ref[...] = xo
    on_ref[...] = jnp.dot(xo, wn_ref[...], preferred_element_type=jnp.float32)


def _layer(x, hpart, ws, wnext, b, n):
    return pl.pallas_call(
        _layer_body,
        grid=(n // ROWB,),
        in_specs=[
            pl.BlockSpec((ROWB, HID), lambda i: (i, 0)),
            pl.BlockSpec((None, ROWB, HID), lambda i: (0, i, 0)),
            pl.BlockSpec((None, ROWB, HID), lambda i: (1, i, 0)),
            pl.BlockSpec((HID, HID), lambda i: (0, 0)),
            pl.BlockSpec((HID, HID), lambda i: (0, 0)),
            pl.BlockSpec((1, HID), lambda i: (0, 0)),
        ],
        out_specs=[pl.BlockSpec((ROWB, HID), lambda i: (i, 0)),
                   pl.BlockSpec((ROWB, HID), lambda i: (i, 0))],
        out_shape=[jax.ShapeDtypeStruct((n, HID), jnp.float32),
                   jax.ShapeDtypeStruct((n, HID), jnp.float32)],
    )(x, hpart, hpart, ws, wnext, b.reshape(1, HID))


def _prep2_body(x_ref, xn_ref, bt_ref, wb_ref, d_ref, a_ref, b_ref,
                ps_ref, pc_ref, acc_s, acc_c):
    i = pl.program_id(0)
    x = x_ref[...]
    bt = bt_ref[...]
    ohb = (bt[:, :1] == lax.broadcasted_iota(jnp.int32, (ROWB, G), 1)).astype(jnp.float32)
    a_ref[...] = (xn_ref[...]
                  + jnp.dot(ohb, d_ref[...], preferred_element_type=jnp.float32, precision=lax.Precision.HIGHEST))
    b_ref[...] = jnp.dot(x, wb_ref[...], preferred_element_type=jnp.float32)

    @pl.when(i == 0)
    def _():
        acc_s[...] = jnp.zeros_like(acc_s)
        acc_c[...] = jnp.zeros_like(acc_c)

    acc_s[...] += lax.dot_general(ohb, x, (((0,), (0,)), ((), ())),
                                  preferred_element_type=jnp.float32,
                                  precision=lax.Precision.HIGHEST)
    acc_c[...] += jnp.sum(ohb, axis=0).reshape(G, 1)
    ps_ref[...] = acc_s[...]
    pc_ref[...] = acc_c[...]


def _prep2(x, xn, batch, wb, d, n):
    bt = jnp.broadcast_to(batch.reshape(n, 1), (n, G)).astype(jnp.int32)
    return pl.pallas_call(
        _prep2_body,
        grid=(n // ROWB,),
        in_specs=[
            pl.BlockSpec((ROWB, HID), lambda i: (i, 0)),
            pl.BlockSpec((ROWB, HID), lambda i: (i, 0)),
            pl.BlockSpec((ROWB, G), lambda i: (i, 0)),
            pl.BlockSpec((HID, HID), lambda i: (0, 0)),
            pl.BlockSpec((G, HID), lambda i: (0, 0)),
        ],
        out_specs=[
            pl.BlockSpec((ROWB, HID), lambda i: (i, 0)),
            pl.BlockSpec((ROWB, HID), lambda i: (i, 0)),
            pl.BlockSpec((G, HID), lambda i: (0, 0)),
            pl.BlockSpec((G, 1), lambda i: (0, 0)),
        ],
        out_shape=[
            jax.ShapeDtypeStruct((n, HID), jnp.float32),
            jax.ShapeDtypeStruct((n, HID), jnp.float32),
            jax.ShapeDtypeStruct((G, HID), jnp.float32),
            jax.ShapeDtypeStruct((G, 1), jnp.float32),
        ],
        scratch_shapes=[pltpu.VMEM((G, HID), jnp.float32),
                        pltpu.VMEM((G, 1), jnp.float32)],
    )(x, xn, bt, wb, d)


def _passa_body(u_ref, ge_ref, w2_ref, b2_ref, ev_ref, r_ref, base_ref,
                m_ref, cnt_ref, acc_h, acc_m):
    i = pl.program_id(0)

    @pl.when(i == 0)
    def _():
        acc_h[...] = jnp.zeros_like(acc_h)
        acc_m[...] = jnp.full_like(acc_m, -jnp.inf)

    u = u_ref[...]
    ev = jnp.dot(jnp.maximum(u, 0.0), w2_ref[...],
                 preferred_element_type=jnp.float32) + b2_ref[...]
    ev_ref[...] = ev
    ge = ge_ref[0, 0]
    oh = (ge[:, None] == lax.broadcasted_iota(jnp.int32, (ECHUNK, G), 1))
    ohi = oh.astype(jnp.int32)
    base_ref[0] = acc_h[...]
    cums = ohi
    k = 1
    while k < ECHUNK:
        z = jnp.zeros((k, G), jnp.int32)
        cums = cums + jnp.concatenate([z, cums[:ECHUNK - k]], axis=0)
        k *= 2
    rank = jnp.sum((cums - ohi) * ohi, axis=1)
    bsel = jnp.sum(acc_h[...] * ohi, axis=1)
    r_ref[0, 0] = rank + bsel
    acc_h[...] += jnp.sum(ohi, axis=0, keepdims=True)
    evmax = jnp.max(ev, axis=1, keepdims=True)
    colmax = jnp.max(jnp.where(oh, evmax, -jnp.inf), axis=0, keepdims=True)
    acc_m[...] = jnp.maximum(acc_m[...], colmax)
    m_ref[...] = acc_m[...]
    cnt_ref[...] = acc_h[...]


def _passa(u, ge2, w2, b2, e):
    nch = e // ECHUNK
    return pl.pallas_call(
        _passa_body,
        grid=(nch,),
        in_specs=[
            pl.BlockSpec((ECHUNK, HID), lambda i: (i, 0)),
            pl.BlockSpec((1, 1, ECHUNK), lambda i: (i, 0, 0)),
            pl.BlockSpec((HID, EDGE_DIM), lambda i: (0, 0)),
            pl.BlockSpec((1, EDGE_DIM), lambda i: (0, 0)),
        ],
        out_specs=[
            pl.BlockSpec((ECHUNK, EDGE_DIM), lambda i: (i, 0)),
            pl.BlockSpec((1, 1, ECHUNK), lambda i: (i, 0, 0)),
            pl.BlockSpec((1, 1, G), lambda i: (i, 0, 0)),
            pl.BlockSpec((1, G), lambda i: (0, 0)),
            pl.BlockSpec((1, G), lambda i: (0, 0)),
        ],
        out_shape=[
            jax.ShapeDtypeStruct((e, EDGE_DIM), jnp.float32),
            jax.ShapeDtypeStruct((nch, 1, ECHUNK), jnp.int32),
            jax.ShapeDtypeStruct((nch, 1, G), jnp.int32),
            jax.ShapeDtypeStruct((1, G), jnp.float32),
            jax.ShapeDtypeStruct((1, G), jnp.int32),
        ],
        scratch_shapes=[pltpu.VMEM((1, G), jnp.int32),
                        pltpu.VMEM((1, G), jnp.float32)],
    )(u, ge2, w2, b2.reshape(1, EDGE_DIM))


def _passb_body(ev_ref, ge_ref, r_ref, m_ref, es_ref, sv_ref, is_ref,
                ssum_ref, acc_s):
    i = pl.program_id(0)

    @pl.when(i == 0)
    def _():
        acc_s[...] = jnp.zeros_like(acc_s)

    ge = ge_ref[0, 0]
    oh = (ge[:, None] == lax.broadcasted_iota(jnp.int32, (ECHUNK, G), 1))
    ohf = oh.astype(jnp.float32)
    mrow = jnp.sum(ohf * m_ref[...], axis=1, keepdims=True)
    sv = jnp.exp(ev_ref[...] - mrow)
    sv_ref[...] = sv
    esel = jnp.sum(oh.astype(jnp.int32) * es_ref[...], axis=1)
    is_ref[0, 0] = r_ref[0, 0] + esel
    rowsum = jnp.sum(sv, axis=1, keepdims=True)
    acc_s[...] += jnp.sum(jnp.where(oh, rowsum, 0.0), axis=0, keepdims=True)
    ssum_ref[...] = acc_s[...]


def _passb(ev, ge2, r2, m, estart, e):
    nch = e // ECHUNK
    return pl.pallas_call(
        _passb_body,
        grid=(nch,),
        in_specs=[
            pl.BlockSpec((ECHUNK, EDGE_DIM), lambda i: (i, 0)),
            pl.BlockSpec((1, 1, ECHUNK), lambda i: (i, 0, 0)),
            pl.BlockSpec((1, 1, ECHUNK), lambda i: (i, 0, 0)),
            pl.BlockSpec((1, G), lambda i: (0, 0)),
            pl.BlockSpec((1, G), lambda i: (0, 0)),
        ],
        out_specs=[
            pl.BlockSpec((ECHUNK, EDGE_DIM), lambda i: (i, 0)),
            pl.BlockSpec((1, 1, ECHUNK), lambda i: (i, 0, 0)),
            pl.BlockSpec((1, G), lambda i: (0, 0)),
        ],
        out_shape=[
            jax.ShapeDtypeStruct((e, EDGE_DIM), jnp.float32),
            jax.ShapeDtypeStruct((nch, 1, ECHUNK), jnp.int32),
            jax.ShapeDtypeStruct((1, G), jnp.float32),
        ],
        scratch_shapes=[pltpu.VMEM((1, G), jnp.float32)],
    )(ev, ge2, r2, m, estart)


def _tail_body(ps_ref, pc_ref, sfp_ref, wpa_ref, wpb_ref, bp1_ref, wp2_ref,
               o_ref):
    pooled = ps_ref[...] / jnp.maximum(pc_ref[...], 1.0)
    h = jnp.maximum(jnp.dot(pooled, wpa_ref[...], preferred_element_type=jnp.float32)
                    + jnp.dot(sfp_ref[...], wpb_ref[...], preferred_element_type=jnp.float32)
                    + bp1_ref[...], 0.0)
    gv = jnp.dot(h, wp2_ref[...], preferred_element_type=jnp.float32)
    o_ref[...] = jnp.broadcast_to(gv[:, :1], (G, HID))


def _graph_vals(psum, pcnt, sf, wp1, bp1, wp2, bp2):
    wp2p = jnp.zeros((HID, HID), jnp.float32).at[:, 0].set(wp2[:, 0])
    sfp = jnp.zeros((G, HID), jnp.float32).at[:, :STATIC].set(sf)
    wpbp = jnp.zeros((HID, HID), jnp.float32).at[:STATIC].set(wp1[HID:])
    pcb = jnp.broadcast_to(pcnt, (G, HID))
    out = pl.pallas_call(
        _tail_body,
        in_specs=[pl.BlockSpec((G, HID), lambda: (0, 0)),
                  pl.BlockSpec((G, HID), lambda: (0, 0)),
                  pl.BlockSpec((G, HID), lambda: (0, 0)),
                  pl.BlockSpec((HID, HID), lambda: (0, 0)),
                  pl.BlockSpec((HID, HID), lambda: (0, 0)),
                  pl.BlockSpec((1, HID), lambda: (0, 0)),
                  pl.BlockSpec((HID, HID), lambda: (0, 0))],
        out_specs=pl.BlockSpec((G, HID), lambda: (0, 0)),
        out_shape=jax.ShapeDtypeStruct((G, HID), jnp.float32),
    )(psum, pcb, sfp, wp1[:HID], wpbp, bp1.reshape(1, HID), wp2p)
    return out[:, 0] + bp2[0]


def kernel(x, edge_attr, edge_index, batch, static_features,
           emb_node_0, emb_node_1, emb_node_2, emb_node_3,
           emb_edge_0, emb_edge_1, emb_edge_2,
           W_in, b_in,
           Wself_0, Wnbr_0, Wedge_0, b_0,
           Wself_1, Wnbr_1, Wedge_1, b_1,
           Wself_2, Wnbr_2, Wedge_2, b_2,
           We1, be1, We2, be2, Wp1, bp1, Wp2, bp2):
    n = x.shape[0]
    e = edge_attr.shape[0]
    src, dst = edge_index[0], edge_index[1]

    embs = [emb_node_0, emb_node_1, emb_node_2, emb_node_3]
    offs = [0, 40, 50, 58]
    t_all = jnp.zeros((128, EMB), jnp.float32)
    for j in range(4):
        t_all = lax.dynamic_update_slice(t_all, embs[j], (offs[j], 0))
    xh = x.astype(jnp.int32) + jnp.array(offs, jnp.int32)[None, :]

    ii = jnp.arange(192)
    eet = (emb_edge_0[ii // 24] + emb_edge_1[(ii // 4) % 6] + emb_edge_2[ii % 4])
    c_idx = (edge_attr[:, 0] * 24 + edge_attr[:, 1] * 4 + edge_attr[:, 2]).astype(jnp.int32)

    wselfs = [Wself_0, Wself_1, Wself_2]
    wnbrs = [Wnbr_0, Wnbr_1, Wnbr_2]
    wedges = [Wedge_0, Wedge_1, Wedge_2]
    bs = [b_0, b_1, b_2]
    cl = [eet @ wedges[l] for l in range(DEPTH)]

    ne = _emb(xh, t_all, n)
    X = jax.nn.relu(ne @ W_in + b_in)
    Xn = _mm(X, wnbrs[0], n)
    for l in range(DEPTH):
        hsum = jax.ops.segment_sum(Xn[src] + cl[l][c_idx], dst, num_segments=n)
        hpart = jnp.stack([hsum, jnp.zeros_like(hsum)])
        wnext = wnbrs[l + 1] if l + 1 < DEPTH else We1[:HID]
        X, Xn = _layer(X, hpart, wselfs[l], wnext, bs[l], n)

    d_tab = static_features @ We1[2 * HID + EMB:] + be1
    aP, bT, psum, pcnt = _prep2(X, Xn, batch, We1[HID:2 * HID], d_tab, n)
    c3 = eet @ We1[2 * HID:2 * HID + EMB]
    gv = _graph_vals(psum, pcnt, static_features, Wp1, bp1, Wp2, bp2)

    ge = batch[src]
    u = aP[src] + bT[dst] + c3[c_idx]

    ge2 = ge.reshape(e // ECHUNK, 1, ECHUNK)
    ev, r2, bases, mseg, cnt2 = _passa(u, ge2, We2, be2, e)

    cnt = cnt2[0]
    m = jnp.maximum(mseg[0], gv)
    estart = jnp.concatenate([jnp.zeros((1,), jnp.int32), jnp.cumsum(cnt)[:-1]])
    sv, is2, ssum = _passb(ev, ge2, r2, m.reshape(1, G), estart.reshape(1, G), e)

    eg_exp = jnp.exp(gv - m)
    denom = ssum[0] + 2.0 * eg_exp
    scale = 2.0 / denom
    gvals = 2.0 * eg_exp / denom

    isort = is2.reshape(-1)
    s_flat = jnp.zeros((e, EDGE_DIM), jnp.float32).at[isort].set(sv).reshape(-1)
    total = e * EDGE_DIM + 2 * G
    start = 8 * estart + 2 * jnp.arange(G, dtype=jnp.int32)
    gid_of_p = jnp.searchsorted(start, jnp.arange(total), side='right').astype(jnp.int32) - 1
    src_idx = jnp.arange(total) - 2 * gid_of_p
    out = (s_flat[jnp.clip(src_idx, 0, e * EDGE_DIM - 1)]
           * scale[gid_of_p])
    gpos = start + 8 * cnt
    out = out.at[gpos].set(gvals).at[gpos + 1].set(gvals)
    return out


# speedup vs baseline: 5.2766x; 1.1045x over previous
import functools
import jax
import jax.numpy as jnp
from jax import lax
from jax.experimental import pallas as pl
from jax.experimental.pallas import tpu as pltpu
from jax.experimental.pallas import tpu_sc as plsc

HID = 128
EMB = 32
G = 32
STATIC = 16
EDGE_DIM = 8
DEPTH = 3

ROWB = 1000
ECHUNK = 512


def _emb_body(xh_ref, t_ref, o_ref):
    xh = xh_ref[...]
    tab = t_ref[...]
    iota = lax.broadcasted_iota(jnp.int32, (ROWB, 128), 1)
    ne = jnp.dot((xh[:, 0:1] == iota).astype(jnp.float32), tab,
                 preferred_element_type=jnp.float32, precision=lax.Precision.HIGHEST)
    for j in range(1, 4):
        oh = (xh[:, j:j + 1] == iota).astype(jnp.float32)
        ne = ne + jnp.dot(oh, tab, preferred_element_type=jnp.float32,
                          precision=lax.Precision.HIGHEST)
    o_ref[...] = ne


def _emb(xh, t_all, n):
    return pl.pallas_call(
        _emb_body,
        grid=(n // ROWB,),
        in_specs=[
            pl.BlockSpec((ROWB, 4), lambda i: (i, 0)),
            pl.BlockSpec((128, EMB), lambda i: (0, 0)),
        ],
        out_specs=pl.BlockSpec((ROWB, EMB), lambda i: (i, 0)),
        out_shape=jax.ShapeDtypeStruct((n, EMB), jnp.float32),
    )(xh, t_all)


def _mm_body(x_ref, w_ref, o_ref):
    o_ref[...] = jnp.dot(x_ref[...], w_ref[...], preferred_element_type=jnp.float32)


def _mm(x, w, n):
    return pl.pallas_call(
        _mm_body,
        grid=(n // ROWB,),
        in_specs=[pl.BlockSpec((ROWB, HID), lambda i: (i, 0)),
                  pl.BlockSpec((HID, HID), lambda i: (0, 0))],
        out_specs=pl.BlockSpec((ROWB, HID), lambda i: (i, 0)),
        out_shape=jax.ShapeDtypeStruct((n, HID), jnp.float32),
    )(x, w)


def _layer_body(x_ref, h0_ref, h1_ref, ws_ref, wn_ref, b_ref, o_ref, on_ref):
    x = x_ref[...]
    h = h0_ref[...] + h1_ref[...]
    z = (jnp.dot(x, ws_ref[...], preferred_element_type=jnp.float32)
         + h + b_ref[...])
    xo = x + jnp.maximum(z, 0.0)
    o_ref[...] = xo
    on_ref[...] = jnp.dot(xo, wn_ref[...], preferred_element_type=jnp.float32)


def _layer(x, hpart, ws, wnext, b, n):
    return pl.pallas_call(
        _layer_body,
        grid=(n // ROWB,),
        in_specs=[
            pl.BlockSpec((ROWB, HID), lambda i: (i, 0)),
            pl.BlockSpec((None, ROWB, HID), lambda i: (0, i, 0)),
            pl.BlockSpec((None, ROWB, HID), lambda i: (1, i, 0)),
            pl.BlockSpec((HID, HID), lambda i: (0, 0)),
            pl.BlockSpec((HID, HID), lambda i: (0, 0)),
            pl.BlockSpec((1, HID), lambda i: (0, 0)),
        ],
        out_specs=[pl.BlockSpec((ROWB, HID), lambda i: (i, 0)),
                   pl.BlockSpec((ROWB, HID), lambda i: (i, 0))],
        out_shape=[jax.ShapeDtypeStruct((n, HID), jnp.float32),
                   jax.ShapeDtypeStruct((n, HID), jnp.float32)],
    )(x, hpart, hpart, ws, wnext, b.reshape(1, HID))


def _prep2_body(x_ref, xn_ref, bt_ref, wb_ref, d_ref, a_ref, b_ref,
                ps_ref, pc_ref, acc_s, acc_c):
    i = pl.program_id(0)
    x = x_ref[...]
    bt = bt_ref[...]
    ohb = (bt[:, :1] == lax.broadcasted_iota(jnp.int32, (ROWB, G), 1)).astype(jnp.float32)
    a_ref[...] = (xn_ref[...]
                  + jnp.dot(ohb, d_ref[...], preferred_element_type=jnp.float32, precision=lax.Precision.HIGHEST))
    b_ref[...] = jnp.dot(x, wb_ref[...], preferred_element_type=jnp.float32)

    @pl.when(i == 0)
    def _():
        acc_s[...] = jnp.zeros_like(acc_s)
        acc_c[...] = jnp.zeros_like(acc_c)

    acc_s[...] += lax.dot_general(ohb, x, (((0,), (0,)), ((), ())),
                                  preferred_element_type=jnp.float32,
                                  precision=lax.Precision.HIGHEST)
    acc_c[...] += jnp.sum(ohb, axis=0).reshape(G, 1)
    ps_ref[...] = acc_s[...]
    pc_ref[...] = acc_c[...]


def _prep2(x, xn, batch, wb, d, n):
    bt = jnp.broadcast_to(batch.reshape(n, 1), (n, G)).astype(jnp.int32)
    return pl.pallas_call(
        _prep2_body,
        grid=(n // ROWB,),
        in_specs=[
            pl.BlockSpec((ROWB, HID), lambda i: (i, 0)),
            pl.BlockSpec((ROWB, HID), lambda i: (i, 0)),
            pl.BlockSpec((ROWB, G), lambda i: (i, 0)),
            pl.BlockSpec((HID, HID), lambda i: (0, 0)),
            pl.BlockSpec((G, HID), lambda i: (0, 0)),
        ],
        out_specs=[
            pl.BlockSpec((ROWB, HID), lambda i: (i, 0)),
            pl.BlockSpec((ROWB, HID), lambda i: (i, 0)),
            pl.BlockSpec((G, HID), lambda i: (0, 0)),
            pl.BlockSpec((G, 1), lambda i: (0, 0)),
        ],
        out_shape=[
            jax.ShapeDtypeStruct((n, HID), jnp.float32),
            jax.ShapeDtypeStruct((n, HID), jnp.float32),
            jax.ShapeDtypeStruct((G, HID), jnp.float32),
            jax.ShapeDtypeStruct((G, 1), jnp.float32),
        ],
        scratch_shapes=[pltpu.VMEM((G, HID), jnp.float32),
                        pltpu.VMEM((G, 1), jnp.float32)],
    )(x, xn, bt, wb, d)


def _passa_body(u_ref, ge_ref, w2_ref, b2_ref, ev_ref, r_ref, base_ref,
                m_ref, cnt_ref, acc_h, acc_m):
    i = pl.program_id(0)

    @pl.when(i == 0)
    def _():
        acc_h[...] = jnp.zeros_like(acc_h)
        acc_m[...] = jnp.full_like(acc_m, -jnp.inf)

    u = u_ref[...]
    ev = jnp.dot(jnp.maximum(u, 0.0), w2_ref[...],
                 preferred_element_type=jnp.float32) + b2_ref[...]
    ev_ref[...] = ev
    ge = ge_ref[0, 0]
    oh = (ge[:, None] == lax.broadcasted_iota(jnp.int32, (ECHUNK, G), 1))
    ohi = oh.astype(jnp.int32)
    base_ref[0] = acc_h[...]
    cums = ohi
    k = 1
    while k < ECHUNK:
        z = jnp.zeros((k, G), jnp.int32)
        cums = cums + jnp.concatenate([z, cums[:ECHUNK - k]], axis=0)
        k *= 2
    rank = jnp.sum((cums - ohi) * ohi, axis=1)
    bsel = jnp.sum(acc_h[...] * ohi, axis=1)
    r_ref[0, 0] = rank + bsel
    acc_h[...] += jnp.sum(ohi, axis=0, keepdims=True)
    evmax = jnp.max(ev, axis=1, keepdims=True)
    colmax = jnp.max(jnp.where(oh, evmax, -jnp.inf), axis=0, keepdims=True)
    acc_m[...] = jnp.maximum(acc_m[...], colmax)
    m_ref[...] = acc_m[...]
    cnt_ref[...] = acc_h[...]


def _passa(u, ge2, w2, b2, e):
    nch = e // ECHUNK
    return pl.pallas_call(
        _passa_body,
        grid=(nch,),
        in_specs=[
            pl.BlockSpec((ECHUNK, HID), lambda i: (i, 0)),
            pl.BlockSpec((1, 1, ECHUNK), lambda i: (i, 0, 0)),
            pl.BlockSpec((HID, EDGE_DIM), lambda i: (0, 0)),
            pl.BlockSpec((1, EDGE_DIM), lambda i: (0, 0)),
        ],
        out_specs=[
            pl.BlockSpec((ECHUNK, EDGE_DIM), lambda i: (i, 0)),
            pl.BlockSpec((1, 1, ECHUNK), lambda i: (i, 0, 0)),
            pl.BlockSpec((1, 1, G), lambda i: (i, 0, 0)),
            pl.BlockSpec((1, G), lambda i: (0, 0)),
            pl.BlockSpec((1, G), lambda i: (0, 0)),
        ],
        out_shape=[
            jax.ShapeDtypeStruct((e, EDGE_DIM), jnp.float32),
            jax.ShapeDtypeStruct((nch, 1, ECHUNK), jnp.int32),
            jax.ShapeDtypeStruct((nch, 1, G), jnp.int32),
            jax.ShapeDtypeStruct((1, G), jnp.float32),
            jax.ShapeDtypeStruct((1, G), jnp.int32),
        ],
        scratch_shapes=[pltpu.VMEM((1, G), jnp.int32),
                        pltpu.VMEM((1, G), jnp.float32)],
    )(u, ge2, w2, b2.reshape(1, EDGE_DIM))


def _passb_body(ev_ref, ge_ref, r_ref, m_ref, es_ref, sv_ref, is_ref,
                ssum_ref, acc_s):
    i = pl.program_id(0)

    @pl.when(i == 0)
    def _():
        acc_s[...] = jnp.zeros_like(acc_s)

    ge = ge_ref[0, 0]
    oh = (ge[:, None] == lax.broadcasted_iota(jnp.int32, (ECHUNK, G), 1))
    ohf = oh.astype(jnp.float32)
    mrow = jnp.sum(ohf * m_ref[...], axis=1, keepdims=True)
    sv = jnp.exp(ev_ref[...] - mrow)
    sv_ref[...] = sv
    esel = jnp.sum(oh.astype(jnp.int32) * es_ref[...], axis=1)
    is_ref[0, 0] = r_ref[0, 0] + esel
    rowsum = jnp.sum(sv, axis=1, keepdims=True)
    acc_s[...] += jnp.sum(jnp.where(oh, rowsum, 0.0), axis=0, keepdims=True)
    ssum_ref[...] = acc_s[...]


def _passb(ev, ge2, r2, m, estart, e):
    nch = e // ECHUNK
    return pl.pallas_call(
        _passb_body,
        grid=(nch,),
        in_specs=[
            pl.BlockSpec((ECHUNK, EDGE_DIM), lambda i: (i, 0)),
            pl.BlockSpec((1, 1, ECHUNK), lambda i: (i, 0, 0)),
            pl.BlockSpec((1, 1, ECHUNK), lambda i: (i, 0, 0)),
            pl.BlockSpec((1, G), lambda i: (0, 0)),
            pl.BlockSpec((1, G), lambda i: (0, 0)),
        ],
        out_specs=[
            pl.BlockSpec((ECHUNK, EDGE_DIM), lambda i: (i, 0)),
            pl.BlockSpec((1, 1, ECHUNK), lambda i: (i, 0, 0)),
            pl.BlockSpec((1, G), lambda i: (0, 0)),
        ],
        out_shape=[
            jax.ShapeDtypeStruct((e, EDGE_DIM), jnp.float32),
            jax.ShapeDtypeStruct((nch, 1, ECHUNK), jnp.int32),
            jax.ShapeDtypeStruct((1, G), jnp.float32),
        ],
        scratch_shapes=[pltpu.VMEM((1, G), jnp.float32)],
    )(ev, ge2, r2, m, estart)


def _tail_body(ps_ref, pc_ref, sfp_ref, wpa_ref, wpb_ref, bp1_ref, wp2_ref,
               o_ref):
    pooled = ps_ref[...] / jnp.maximum(pc_ref[...], 1.0)
    h = jnp.maximum(jnp.dot(pooled, wpa_ref[...], preferred_element_type=jnp.float32)
                    + jnp.dot(sfp_ref[...], wpb_ref[...], preferred_element_type=jnp.float32)
                    + bp1_ref[...], 0.0)
    gv = jnp.dot(h, wp2_ref[...], preferred_element_type=jnp.float32)
    o_ref[...] = jnp.broadcast_to(gv[:, :1], (G, HID))


def _graph_vals(psum, pcnt, sf, wp1, bp1, wp2, bp2):
    wp2p = jnp.zeros((HID, HID), jnp.float32).at[:, 0].set(wp2[:, 0])
    sfp = jnp.zeros((G, HID), jnp.float32).at[:, :STATIC].set(sf)
    wpbp = jnp.zeros((HID, HID), jnp.float32).at[:STATIC].set(wp1[HID:])
    pcb = jnp.broadcast_to(pcnt, (G, HID))
    out = pl.pallas_call(
        _tail_body,
        in_specs=[pl.BlockSpec((G, HID), lambda: (0, 0)),
                  pl.BlockSpec((G, HID), lambda: (0, 0)),
                  pl.BlockSpec((G, HID), lambda: (0, 0)),
                  pl.BlockSpec((HID, HID), lambda: (0, 0)),
                  pl.BlockSpec((HID, HID), lambda: (0, 0)),
                  pl.BlockSpec((1, HID), lambda: (0, 0)),
                  pl.BlockSpec((HID, HID), lambda: (0, 0))],
        out_specs=pl.BlockSpec((G, HID), lambda: (0, 0)),
        out_shape=jax.ShapeDtypeStruct((G, HID), jnp.float32),
    )(psum, pcb, sfp, wp1[:HID], wpbp, bp1.reshape(1, HID), wp2p)
    return out[:, 0] + bp2[0]


SCK = 80


def _sc_gather3(a, b, c3, si, di, ci, e):
    info = plsc.get_sparse_core_info()
    nc, ns = info.num_cores, info.num_subcores
    nw = nc * ns
    per_w = e // nw
    nch = per_w // SCK
    mesh = plsc.VectorSubcoreMesh(core_axis_name="c", subcore_axis_name="s")

    @functools.partial(
        pl.kernel, mesh=mesh,
        out_type=jax.ShapeDtypeStruct((e, HID), jnp.float32),
        scratch_types=[
            pltpu.VMEM((1, SCK), jnp.int32),
            pltpu.VMEM((1, SCK), jnp.int32),
            pltpu.VMEM((1, SCK), jnp.int32),
            pltpu.VMEM((SCK, HID), jnp.float32),
            pltpu.VMEM((SCK, HID), jnp.float32),
            pltpu.VMEM((SCK, HID), jnp.float32),
            pltpu.SemaphoreType.DMA,
        ],
    )
    def k(a_hbm, b_hbm, c_hbm, si_hbm, di_hbm, ci_hbm, u_hbm,
          siv, div, civ, r0, r1, r2, sem):
        wid = lax.axis_index("s") * nc + lax.axis_index("c")
        base = wid * per_w

        def body(i, carry):
            off = base + i * SCK
            pltpu.sync_copy(si_hbm.at[pl.ds(off, SCK)], siv.at[0])
            pltpu.sync_copy(di_hbm.at[pl.ds(off, SCK)], div.at[0])
            pltpu.sync_copy(ci_hbm.at[pl.ds(off, SCK)], civ.at[0])
            cp0 = pltpu.async_copy(a_hbm.at[siv.at[0]], r0, sem)
            cp1 = pltpu.async_copy(b_hbm.at[div.at[0]], r1, sem)
            cp2 = pltpu.async_copy(c_hbm.at[civ.at[0]], r2, sem)
            cp0.wait()
            cp1.wait()
            cp2.wait()

            def addrow(r, _):
                for c in range(HID // 16):
                    s = pl.ds(c * 16, 16)
                    r0[r, s] = r0[r, s] + r1[r, s] + r2[r, s]
                return 0

            lax.fori_loop(0, SCK, addrow, 0)
            pltpu.sync_copy(r0, u_hbm.at[pl.ds(off, SCK)])
            return carry

        lax.fori_loop(0, nch, body, 0)

    return k(a, b, c3, si, di, ci)


def kernel(x, edge_attr, edge_index, batch, static_features,
           emb_node_0, emb_node_1, emb_node_2, emb_node_3,
           emb_edge_0, emb_edge_1, emb_edge_2,
           W_in, b_in,
           Wself_0, Wnbr_0, Wedge_0, b_0,
           Wself_1, Wnbr_1, Wedge_1, b_1,
           Wself_2, Wnbr_2, Wedge_2, b_2,
           We1, be1, We2, be2, Wp1, bp1, Wp2, bp2):
    n = x.shape[0]
    e = edge_attr.shape[0]
    src, dst = edge_index[0], edge_index[1]

    embs = [emb_node_0, emb_node_1, emb_node_2, emb_node_3]
    offs = [0, 40, 50, 58]
    t_all = jnp.zeros((128, EMB), jnp.float32)
    for j in range(4):
        t_all = lax.dynamic_update_slice(t_all, embs[j], (offs[j], 0))
    xh = x.astype(jnp.int32) + jnp.array(offs, jnp.int32)[None, :]

    ii = jnp.arange(192)
    eet = (emb_edge_0[ii // 24] + emb_edge_1[(ii // 4) % 6] + emb_edge_2[ii % 4])
    c_idx = (edge_attr[:, 0] * 24 + edge_attr[:, 1] * 4 + edge_attr[:, 2]).astype(jnp.int32)

    wselfs = [Wself_0, Wself_1, Wself_2]
    wnbrs = [Wnbr_0, Wnbr_1, Wnbr_2]
    wedges = [Wedge_0, Wedge_1, Wedge_2]
    bs = [b_0, b_1, b_2]
    cl = [eet @ wedges[l] for l in range(DEPTH)]

    ne = _emb(xh, t_all, n)
    X = jax.nn.relu(ne @ W_in + b_in)
    Xn = _mm(X, wnbrs[0], n)
    for l in range(DEPTH):
        hsum = jax.ops.segment_sum(Xn[src] + cl[l][c_idx], dst, num_segments=n)
        hpart = jnp.stack([hsum, jnp.zeros_like(hsum)])
        wnext = wnbrs[l + 1] if l + 1 < DEPTH else We1[:HID]
        X, Xn = _layer(X, hpart, wselfs[l], wnext, bs[l], n)

    d_tab = static_features @ We1[2 * HID + EMB:] + be1
    aP, bT, psum, pcnt = _prep2(X, Xn, batch, We1[HID:2 * HID], d_tab, n)
    c3 = eet @ We1[2 * HID:2 * HID + EMB]
    gv = _graph_vals(psum, pcnt, static_features, Wp1, bp1, Wp2, bp2)

    ge = batch[src]
    u = _sc_gather3(aP, bT, c3, src.astype(jnp.int32), dst.astype(jnp.int32),
                    c_idx, e)

    ge2 = ge.reshape(e // ECHUNK, 1, ECHUNK)
    ev, r2, bases, mseg, cnt2 = _passa(u, ge2, We2, be2, e)

    cnt = cnt2[0]
    m = jnp.maximum(mseg[0], gv)
    estart = jnp.concatenate([jnp.zeros((1,), jnp.int32), jnp.cumsum(cnt)[:-1]])
    sv, is2, ssum = _passb(ev, ge2, r2, m.reshape(1, G), estart.reshape(1, G), e)

    eg_exp = jnp.exp(gv - m)
    denom = ssum[0] + 2.0 * eg_exp
    scale = 2.0 / denom
    gvals = 2.0 * eg_exp / denom

    isort = is2.reshape(-1)
    s_flat = jnp.zeros((e, EDGE_DIM), jnp.float32).at[isort].set(sv).reshape(-1)
    total = e * EDGE_DIM + 2 * G
    start = 8 * estart + 2 * jnp.arange(G, dtype=jnp.int32)
    gid_of_p = jnp.searchsorted(start, jnp.arange(total), side='right').astype(jnp.int32) - 1
    src_idx = jnp.arange(total) - 2 * gid_of_p
    out = (s_flat[jnp.clip(src_idx, 0, e * EDGE_DIM - 1)]
           * scale[gid_of_p])
    gpos = start + 8 * cnt
    out = out.at[gpos].set(gvals).at[gpos + 1].set(gvals)
    return out


# speedup vs baseline: 7.3733x; 1.3974x over previous
import functools
import jax
import jax.numpy as jnp
from jax import lax
from jax.experimental import pallas as pl
from jax.experimental.pallas import tpu as pltpu
from jax.experimental.pallas import tpu_sc as plsc

HID = 128
EMB = 32
G = 32
STATIC = 16
EDGE_DIM = 8
DEPTH = 3

ROWB = 1000
ECHUNK = 512


def _emb_body(xh_ref, t_ref, o_ref):
    xh = xh_ref[...]
    tab = t_ref[...]
    iota = lax.broadcasted_iota(jnp.int32, (ROWB, 128), 1)
    ne = jnp.dot((xh[:, 0:1] == iota).astype(jnp.float32), tab,
                 preferred_element_type=jnp.float32, precision=lax.Precision.HIGHEST)
    for j in range(1, 4):
        oh = (xh[:, j:j + 1] == iota).astype(jnp.float32)
        ne = ne + jnp.dot(oh, tab, preferred_element_type=jnp.float32,
                          precision=lax.Precision.HIGHEST)
    o_ref[...] = ne


def _emb(xh, t_all, n):
    return pl.pallas_call(
        _emb_body,
        grid=(n // ROWB,),
        in_specs=[
            pl.BlockSpec((ROWB, 4), lambda i: (i, 0)),
            pl.BlockSpec((128, EMB), lambda i: (0, 0)),
        ],
        out_specs=pl.BlockSpec((ROWB, EMB), lambda i: (i, 0)),
        out_shape=jax.ShapeDtypeStruct((n, EMB), jnp.float32),
    )(xh, t_all)


def _mm_body(x_ref, w_ref, o_ref):
    o_ref[...] = jnp.dot(x_ref[...], w_ref[...], preferred_element_type=jnp.float32)


def _mm(x, w, n):
    return pl.pallas_call(
        _mm_body,
        grid=(n // ROWB,),
        in_specs=[pl.BlockSpec((ROWB, HID), lambda i: (i, 0)),
                  pl.BlockSpec((HID, HID), lambda i: (0, 0))],
        out_specs=pl.BlockSpec((ROWB, HID), lambda i: (i, 0)),
        out_shape=jax.ShapeDtypeStruct((n, HID), jnp.float32),
    )(x, w)


def _layer_body(x_ref, h0_ref, h1_ref, ws_ref, wn_ref, b_ref, o_ref, on_ref):
    x = x_ref[...]
    h = h0_ref[...] + h1_ref[...]
    z = (jnp.dot(x, ws_ref[...], preferred_element_type=jnp.float32)
         + h + b_ref[...])
    xo = x + jnp.maximum(z, 0.0)
    o_ref[...] = xo
    on_ref[...] = jnp.dot(xo, wn_ref[...], preferred_element_type=jnp.float32)


def _layer(x, hpart, ws, wnext, b, n):
    return pl.pallas_call(
        _layer_body,
        grid=(n // ROWB,),
        in_specs=[
            pl.BlockSpec((ROWB, HID), lambda i: (i, 0)),
            pl.BlockSpec((None, ROWB, HID), lambda i: (0, i, 0)),
            pl.BlockSpec((None, ROWB, HID), lambda i: (1, i, 0)),
            pl.BlockSpec((HID, HID), lambda i: (0, 0)),
            pl.BlockSpec((HID, HID), lambda i: (0, 0)),
            pl.BlockSpec((1, HID), lambda i: (0, 0)),
        ],
        out_specs=[pl.BlockSpec((ROWB, HID), lambda i: (i, 0)),
                   pl.BlockSpec((ROWB, HID), lambda i: (i, 0))],
        out_shape=[jax.ShapeDtypeStruct((n, HID), jnp.float32),
                   jax.ShapeDtypeStruct((n, HID), jnp.float32)],
    )(x, hpart, hpart, ws, wnext, b.reshape(1, HID))


def _prep2_body(x_ref, xn_ref, bt_ref, wb_ref, d_ref, a_ref, b_ref,
                ps_ref, pc_ref, acc_s, acc_c):
    i = pl.program_id(0)
    x = x_ref[...]
    bt = bt_ref[...]
    ohb = (bt[:, :1] == lax.broadcasted_iota(jnp.int32, (ROWB, G), 1)).astype(jnp.float32)
    a_ref[...] = (xn_ref[...]
                  + jnp.dot(ohb, d_ref[...], preferred_element_type=jnp.float32, precision=lax.Precision.HIGHEST))
    b_ref[...] = jnp.dot(x, wb_ref[...], preferred_element_type=jnp.float32)

    @pl.when(i == 0)
    def _():
        acc_s[...] = jnp.zeros_like(acc_s)
        acc_c[...] = jnp.zeros_like(acc_c)

    acc_s[...] += lax.dot_general(ohb, x, (((0,), (0,)), ((), ())),
                                  preferred_element_type=jnp.float32,
                                  precision=lax.Precision.HIGHEST)
    acc_c[...] += jnp.sum(ohb, axis=0).reshape(G, 1)
    ps_ref[...] = acc_s[...]
    pc_ref[...] = acc_c[...]


def _prep2(x, xn, batch, wb, d, n):
    bt = jnp.broadcast_to(batch.reshape(n, 1), (n, G)).astype(jnp.int32)
    return pl.pallas_call(
        _prep2_body,
        grid=(n // ROWB,),
        in_specs=[
            pl.BlockSpec((ROWB, HID), lambda i: (i, 0)),
            pl.BlockSpec((ROWB, HID), lambda i: (i, 0)),
            pl.BlockSpec((ROWB, G), lambda i: (i, 0)),
            pl.BlockSpec((HID, HID), lambda i: (0, 0)),
            pl.BlockSpec((G, HID), lambda i: (0, 0)),
        ],
        out_specs=[
            pl.BlockSpec((ROWB, HID), lambda i: (i, 0)),
            pl.BlockSpec((ROWB, HID), lambda i: (i, 0)),
            pl.BlockSpec((G, HID), lambda i: (0, 0)),
            pl.BlockSpec((G, 1), lambda i: (0, 0)),
        ],
        out_shape=[
            jax.ShapeDtypeStruct((n, HID), jnp.float32),
            jax.ShapeDtypeStruct((n, HID), jnp.float32),
            jax.ShapeDtypeStruct((G, HID), jnp.float32),
            jax.ShapeDtypeStruct((G, 1), jnp.float32),
        ],
        scratch_shapes=[pltpu.VMEM((G, HID), jnp.float32),
                        pltpu.VMEM((G, 1), jnp.float32)],
    )(x, xn, bt, wb, d)


def _passa_body(u_ref, ge_ref, w2_ref, b2_ref, ev_ref, r_ref, base_ref,
                m_ref, cnt_ref, acc_h, acc_m):
    i = pl.program_id(0)

    @pl.when(i == 0)
    def _():
        acc_h[...] = jnp.zeros_like(acc_h)
        acc_m[...] = jnp.full_like(acc_m, -jnp.inf)

    u = u_ref[...]
    ev = jnp.dot(jnp.maximum(u, 0.0), w2_ref[...],
                 preferred_element_type=jnp.float32) + b2_ref[...]
    ev_ref[...] = ev
    ge = ge_ref[0, 0]
    oh = (ge[:, None] == lax.broadcasted_iota(jnp.int32, (ECHUNK, G), 1))
    ohi = oh.astype(jnp.int32)
    base_ref[0] = acc_h[...]
    cums = ohi
    k = 1
    while k < ECHUNK:
        z = jnp.zeros((k, G), jnp.int32)
        cums = cums + jnp.concatenate([z, cums[:ECHUNK - k]], axis=0)
        k *= 2
    rank = jnp.sum((cums - ohi) * ohi, axis=1)
    bsel = jnp.sum(acc_h[...] * ohi, axis=1)
    r_ref[0, 0] = rank + bsel
    acc_h[...] += jnp.sum(ohi, axis=0, keepdims=True)
    evmax = jnp.max(ev, axis=1, keepdims=True)
    colmax = jnp.max(jnp.where(oh, evmax, -jnp.inf), axis=0, keepdims=True)
    acc_m[...] = jnp.maximum(acc_m[...], colmax)
    m_ref[...] = acc_m[...]
    cnt_ref[...] = acc_h[...]


def _passa(u, ge2, w2, b2, e):
    nch = e // ECHUNK
    return pl.pallas_call(
        _passa_body,
        grid=(nch,),
        in_specs=[
            pl.BlockSpec((ECHUNK, HID), lambda i: (i, 0)),
            pl.BlockSpec((1, 1, ECHUNK), lambda i: (i, 0, 0)),
            pl.BlockSpec((HID, EDGE_DIM), lambda i: (0, 0)),
            pl.BlockSpec((1, EDGE_DIM), lambda i: (0, 0)),
        ],
        out_specs=[
            pl.BlockSpec((ECHUNK, EDGE_DIM), lambda i: (i, 0)),
            pl.BlockSpec((1, 1, ECHUNK), lambda i: (i, 0, 0)),
            pl.BlockSpec((1, 1, G), lambda i: (i, 0, 0)),
            pl.BlockSpec((1, G), lambda i: (0, 0)),
            pl.BlockSpec((1, G), lambda i: (0, 0)),
        ],
        out_shape=[
            jax.ShapeDtypeStruct((e, EDGE_DIM), jnp.float32),
            jax.ShapeDtypeStruct((nch, 1, ECHUNK), jnp.int32),
            jax.ShapeDtypeStruct((nch, 1, G), jnp.int32),
            jax.ShapeDtypeStruct((1, G), jnp.float32),
            jax.ShapeDtypeStruct((1, G), jnp.int32),
        ],
        scratch_shapes=[pltpu.VMEM((1, G), jnp.int32),
                        pltpu.VMEM((1, G), jnp.float32)],
    )(u, ge2, w2, b2.reshape(1, EDGE_DIM))


def _passb_body(ev_ref, ge_ref, r_ref, m_ref, es_ref, sv_ref, is_ref,
                ssum_ref, acc_s):
    i = pl.program_id(0)

    @pl.when(i == 0)
    def _():
        acc_s[...] = jnp.zeros_like(acc_s)

    ge = ge_ref[0, 0]
    oh = (ge[:, None] == lax.broadcasted_iota(jnp.int32, (ECHUNK, G), 1))
    ohf = oh.astype(jnp.float32)
    mrow = jnp.sum(ohf * m_ref[...], axis=1, keepdims=True)
    sv = jnp.exp(ev_ref[...] - mrow)
    sv_ref[...] = sv
    esel = jnp.sum(oh.astype(jnp.int32) * es_ref[...], axis=1)
    is_ref[0, 0] = r_ref[0, 0] + esel
    rowsum = jnp.sum(sv, axis=1, keepdims=True)
    acc_s[...] += jnp.sum(jnp.where(oh, rowsum, 0.0), axis=0, keepdims=True)
    ssum_ref[...] = acc_s[...]


def _passb(ev, ge2, r2, m, estart, e):
    nch = e // ECHUNK
    return pl.pallas_call(
        _passb_body,
        grid=(nch,),
        in_specs=[
            pl.BlockSpec((ECHUNK, EDGE_DIM), lambda i: (i, 0)),
            pl.BlockSpec((1, 1, ECHUNK), lambda i: (i, 0, 0)),
            pl.BlockSpec((1, 1, ECHUNK), lambda i: (i, 0, 0)),
            pl.BlockSpec((1, G), lambda i: (0, 0)),
            pl.BlockSpec((1, G), lambda i: (0, 0)),
        ],
        out_specs=[
            pl.BlockSpec((ECHUNK, EDGE_DIM), lambda i: (i, 0)),
            pl.BlockSpec((1, 1, ECHUNK), lambda i: (i, 0, 0)),
            pl.BlockSpec((1, G), lambda i: (0, 0)),
        ],
        out_shape=[
            jax.ShapeDtypeStruct((e, EDGE_DIM), jnp.float32),
            jax.ShapeDtypeStruct((nch, 1, ECHUNK), jnp.int32),
            jax.ShapeDtypeStruct((1, G), jnp.float32),
        ],
        scratch_shapes=[pltpu.VMEM((1, G), jnp.float32)],
    )(ev, ge2, r2, m, estart)


def _tail_body(ps_ref, pc_ref, sfp_ref, wpa_ref, wpb_ref, bp1_ref, wp2_ref,
               o_ref):
    pooled = ps_ref[...] / jnp.maximum(pc_ref[...], 1.0)
    h = jnp.maximum(jnp.dot(pooled, wpa_ref[...], preferred_element_type=jnp.float32)
                    + jnp.dot(sfp_ref[...], wpb_ref[...], preferred_element_type=jnp.float32)
                    + bp1_ref[...], 0.0)
    gv = jnp.dot(h, wp2_ref[...], preferred_element_type=jnp.float32)
    o_ref[...] = jnp.broadcast_to(gv[:, :1], (G, HID))


def _graph_vals(psum, pcnt, sf, wp1, bp1, wp2, bp2):
    wp2p = jnp.zeros((HID, HID), jnp.float32).at[:, 0].set(wp2[:, 0])
    sfp = jnp.zeros((G, HID), jnp.float32).at[:, :STATIC].set(sf)
    wpbp = jnp.zeros((HID, HID), jnp.float32).at[:STATIC].set(wp1[HID:])
    pcb = jnp.broadcast_to(pcnt, (G, HID))
    out = pl.pallas_call(
        _tail_body,
        in_specs=[pl.BlockSpec((G, HID), lambda: (0, 0)),
                  pl.BlockSpec((G, HID), lambda: (0, 0)),
                  pl.BlockSpec((G, HID), lambda: (0, 0)),
                  pl.BlockSpec((HID, HID), lambda: (0, 0)),
                  pl.BlockSpec((HID, HID), lambda: (0, 0)),
                  pl.BlockSpec((1, HID), lambda: (0, 0)),
                  pl.BlockSpec((HID, HID), lambda: (0, 0))],
        out_specs=pl.BlockSpec((G, HID), lambda: (0, 0)),
        out_shape=jax.ShapeDtypeStruct((G, HID), jnp.float32),
    )(psum, pcb, sfp, wp1[:HID], wpbp, bp1.reshape(1, HID), wp2p)
    return out[:, 0] + bp2[0]


SCK = 80


def _sc_gather3(a, b, c3, si, di, ci, e):
    info = plsc.get_sparse_core_info()
    nc, ns = info.num_cores, info.num_subcores
    nw = nc * ns
    per_w = e // nw
    nch = per_w // SCK
    mesh = plsc.VectorSubcoreMesh(core_axis_name="c", subcore_axis_name="s")

    @functools.partial(
        pl.kernel, mesh=mesh,
        out_type=jax.ShapeDtypeStruct((e, HID), jnp.float32),
        scratch_types=[
            pltpu.VMEM((1, SCK), jnp.int32),
            pltpu.VMEM((1, SCK), jnp.int32),
            pltpu.VMEM((1, SCK), jnp.int32),
            pltpu.VMEM((SCK, HID), jnp.float32),
            pltpu.VMEM((SCK, HID), jnp.float32),
            pltpu.VMEM((SCK, HID), jnp.float32),
            pltpu.SemaphoreType.DMA,
        ],
    )
    def k(a_hbm, b_hbm, c_hbm, si_hbm, di_hbm, ci_hbm, u_hbm,
          siv, div, civ, r0, r1, r2, sem):
        wid = lax.axis_index("s") * nc + lax.axis_index("c")
        base = wid * per_w

        def body(i, carry):
            off = base + i * SCK
            pltpu.sync_copy(si_hbm.at[pl.ds(off, SCK)], siv.at[0])
            pltpu.sync_copy(di_hbm.at[pl.ds(off, SCK)], div.at[0])
            pltpu.sync_copy(ci_hbm.at[pl.ds(off, SCK)], civ.at[0])
            cp0 = pltpu.async_copy(a_hbm.at[siv.at[0]], r0, sem)
            cp1 = pltpu.async_copy(b_hbm.at[div.at[0]], r1, sem)
            cp2 = pltpu.async_copy(c_hbm.at[civ.at[0]], r2, sem)
            cp0.wait()
            cp1.wait()
            cp2.wait()

            def addrow(r, _):
                for c in range(HID // 16):
                    s = pl.ds(c * 16, 16)
                    r0[r, s] = r0[r, s] + r1[r, s] + r2[r, s]
                return 0

            lax.fori_loop(0, SCK, addrow, 0)
            pltpu.sync_copy(r0, u_hbm.at[pl.ds(off, SCK)])
            return carry

        lax.fori_loop(0, nch, body, 0)

    return k(a, b, c3, si, di, ci)


def _sc_segsum(xn, cl, si, di, ci, e, n):
    info = plsc.get_sparse_core_info()
    nc, ns = info.num_cores, info.num_subcores
    per_w = e // (nc * ns)
    nch = per_w // SCK
    nzch = (n + SCK - 1) // SCK
    mesh = plsc.VectorSubcoreMesh(core_axis_name="c", subcore_axis_name="s")

    @functools.partial(
        pl.kernel, mesh=mesh,
        out_type=jax.ShapeDtypeStruct((2, n, HID), jnp.float32),
        scratch_types=[
            pltpu.VMEM((1, SCK), jnp.int32),
            pltpu.VMEM((1, SCK), jnp.int32),
            pltpu.VMEM((1, SCK), jnp.int32),
            pltpu.VMEM((SCK, HID), jnp.float32),
            pltpu.VMEM((SCK, HID), jnp.float32),
            pltpu.VMEM_SHARED((n, HID), jnp.float32),
            pltpu.SemaphoreType.DMA,
        ],
    )
    def k(xn_hbm, cl_hbm, si_hbm, di_hbm, ci_hbm, hp_hbm,
          siv, div, civ, r0, r1, hacc, sem):
        cid = lax.axis_index("c")
        sid = lax.axis_index("s")
        wid = sid * nc + cid
        base = wid * per_w

        def zrow(r, _):
            for c in range(HID // 16):
                r0[r, pl.ds(c * 16, 16)] = jnp.zeros((16,), jnp.float32)
            return 0

        lax.fori_loop(0, SCK, zrow, 0)

        def zchunk(j, _):
            pltpu.sync_copy(r0, hacc.at[pl.ds(j * SCK, SCK)])
            return 0

        lax.fori_loop(0, nzch // ns + 1, lambda t, _:
                      lax.cond(t * ns + sid < nzch,
                               lambda: zchunk(t * ns + sid, 0),
                               lambda: 0), 0)
        plsc.subcore_barrier()

        def body(i, carry):
            off = base + i * SCK
            pltpu.sync_copy(si_hbm.at[pl.ds(off, SCK)], siv.at[0])
            pltpu.sync_copy(di_hbm.at[pl.ds(off, SCK)], div.at[0])
            pltpu.sync_copy(ci_hbm.at[pl.ds(off, SCK)], civ.at[0])
            cp0 = pltpu.async_copy(xn_hbm.at[siv.at[0]], r0, sem)
            cp1 = pltpu.async_copy(cl_hbm.at[civ.at[0]], r1, sem)
            cp0.wait()
            cp1.wait()

            def addrow(r, _):
                for c in range(HID // 16):
                    s = pl.ds(c * 16, 16)
                    r0[r, s] = r0[r, s] + r1[r, s]
                return 0

            lax.fori_loop(0, SCK, addrow, 0)
            pltpu.sync_copy(r0, hacc.at[div.at[0]], add=True)
            return carry

        lax.fori_loop(0, nch, body, 0)
        plsc.subcore_barrier()

        def wchunk(j, _):
            pltpu.sync_copy(hacc.at[pl.ds(j * SCK, SCK)], r0)
            pltpu.sync_copy(r0, hp_hbm.at[cid, pl.ds(j * SCK, SCK)])
            return 0

        lax.fori_loop(0, nzch // ns + 1, lambda t, _:
                      lax.cond(t * ns + sid < nzch,
                               lambda: wchunk(t * ns + sid, 0),
                               lambda: 0), 0)

    return k(xn, cl, si, di, ci)


def kernel(x, edge_attr, edge_index, batch, static_features,
           emb_node_0, emb_node_1, emb_node_2, emb_node_3,
           emb_edge_0, emb_edge_1, emb_edge_2,
           W_in, b_in,
           Wself_0, Wnbr_0, Wedge_0, b_0,
           Wself_1, Wnbr_1, Wedge_1, b_1,
           Wself_2, Wnbr_2, Wedge_2, b_2,
           We1, be1, We2, be2, Wp1, bp1, Wp2, bp2):
    n = x.shape[0]
    e = edge_attr.shape[0]
    src, dst = edge_index[0], edge_index[1]

    embs = [emb_node_0, emb_node_1, emb_node_2, emb_node_3]
    offs = [0, 40, 50, 58]
    t_all = jnp.zeros((128, EMB), jnp.float32)
    for j in range(4):
        t_all = lax.dynamic_update_slice(t_all, embs[j], (offs[j], 0))
    xh = x.astype(jnp.int32) + jnp.array(offs, jnp.int32)[None, :]

    ii = jnp.arange(192)
    eet = (emb_edge_0[ii // 24] + emb_edge_1[(ii // 4) % 6] + emb_edge_2[ii % 4])
    c_idx = (edge_attr[:, 0] * 24 + edge_attr[:, 1] * 4 + edge_attr[:, 2]).astype(jnp.int32)

    wselfs = [Wself_0, Wself_1, Wself_2]
    wnbrs = [Wnbr_0, Wnbr_1, Wnbr_2]
    wedges = [Wedge_0, Wedge_1, Wedge_2]
    bs = [b_0, b_1, b_2]
    cl = [eet @ wedges[l] for l in range(DEPTH)]

    ne = _emb(xh, t_all, n)
    X = jax.nn.relu(ne @ W_in + b_in)
    Xn = _mm(X, wnbrs[0], n)
    srci = src.astype(jnp.int32)
    dsti = dst.astype(jnp.int32)
    for l in range(DEPTH):
        hpart = _sc_segsum(Xn, cl[l], srci, dsti, c_idx, e, n)
        wnext = wnbrs[l + 1] if l + 1 < DEPTH else We1[:HID]
        X, Xn = _layer(X, hpart, wselfs[l], wnext, bs[l], n)

    d_tab = static_features @ We1[2 * HID + EMB:] + be1
    aP, bT, psum, pcnt = _prep2(X, Xn, batch, We1[HID:2 * HID], d_tab, n)
    c3 = eet @ We1[2 * HID:2 * HID + EMB]
    gv = _graph_vals(psum, pcnt, static_features, Wp1, bp1, Wp2, bp2)

    ge = batch[src]
    u = _sc_gather3(aP, bT, c3, srci, dsti, c_idx, e)

    ge2 = ge.reshape(e // ECHUNK, 1, ECHUNK)
    ev, r2, bases, mseg, cnt2 = _passa(u, ge2, We2, be2, e)

    cnt = cnt2[0]
    m = jnp.maximum(mseg[0], gv)
    estart = jnp.concatenate([jnp.zeros((1,), jnp.int32), jnp.cumsum(cnt)[:-1]])
    sv, is2, ssum = _passb(ev, ge2, r2, m.reshape(1, G), estart.reshape(1, G), e)

    eg_exp = jnp.exp(gv - m)
    denom = ssum[0] + 2.0 * eg_exp
    scale = 2.0 / denom
    gvals = 2.0 * eg_exp / denom

    isort = is2.reshape(-1)
    s_flat = jnp.zeros((e, EDGE_DIM), jnp.float32).at[isort].set(sv).reshape(-1)
    total = e * EDGE_DIM + 2 * G
    start = 8 * estart + 2 * jnp.arange(G, dtype=jnp.int32)
    gid_of_p = jnp.searchsorted(start, jnp.arange(total), side='right').astype(jnp.int32) - 1
    src_idx = jnp.arange(total) - 2 * gid_of_p
    out = (s_flat[jnp.clip(src_idx, 0, e * EDGE_DIM - 1)]
           * scale[gid_of_p])
    gpos = start + 8 * cnt
    out = out.at[gpos].set(gvals).at[gpos + 1].set(gvals)
    return out
